# Initial kernel scaffold; baseline (speedup 1.0000x reference)
import jax
import jax.numpy as jnp
from jax.experimental import pallas as pl


def kernel(x, adjacency, emb_table, W, att_src, att_dst, bias):
    raise NotImplementedError("write your pallas kernel here")


# speedup vs baseline: 21.1952x; 21.1952x over previous
import functools

import jax
import jax.numpy as jnp
from jax import lax
from jax.experimental import pallas as pl
from jax.experimental.pallas import tpu as pltpu
from jax.experimental.pallas import tpu_sc as plsc

N = 10000
IN_CH = 128
OUT_CH = 128
HEADS = 2
E = 160000
G = 2

_ROWS_BLK = 2000
_K = 128
_CHUNKS = E // _K
_HC = 64
_NT = 16
_NSL = 80
_NSLICES = N // _NSL


def _tc_proj_kernel(emb_ref, w_ref, asrc_ref, adst_ref, t_out, a_out):
    t = jnp.dot(emb_ref[...], w_ref[...], preferred_element_type=jnp.float32)
    t0 = t[:, :OUT_CH]
    t1 = t[:, OUT_CH:]
    s0 = jnp.sum(t0 * asrc_ref[0:1, :], axis=-1, keepdims=True)
    s1 = jnp.sum(t1 * asrc_ref[1:2, :], axis=-1, keepdims=True)
    d0 = jnp.sum(t0 * adst_ref[0:1, :], axis=-1, keepdims=True)
    d1 = jnp.sum(t1 * adst_ref[1:2, :], axis=-1, keepdims=True)
    t_out[...] = t
    a_out[...] = jnp.concatenate([s0, s1, d0, d1], axis=1)


def _tc_project(emb_table, W, att_src, att_dst):
    return pl.pallas_call(
        _tc_proj_kernel,
        grid=(N // _ROWS_BLK,),
        in_specs=[
            pl.BlockSpec((_ROWS_BLK, IN_CH), lambda i: (i, 0)),
            pl.BlockSpec((IN_CH, HEADS * OUT_CH), lambda i: (0, 0)),
            pl.BlockSpec((HEADS, OUT_CH), lambda i: (0, 0)),
            pl.BlockSpec((HEADS, OUT_CH), lambda i: (0, 0)),
        ],
        out_specs=[
            pl.BlockSpec((_ROWS_BLK, HEADS * OUT_CH), lambda i: (i, 0)),
            pl.BlockSpec((_ROWS_BLK, 4), lambda i: (i, 0)),
        ],
        out_shape=[
            jax.ShapeDtypeStruct((N, HEADS * OUT_CH), jnp.float32),
            jax.ShapeDtypeStruct((N, 4), jnp.float32),
        ],
    )(emb_table, W, att_src, att_dst)


def _sc_gat(x2, src, dst, tq, as0, as1, ad0, ad1):
    mesh = plsc.VectorSubcoreMesh(core_axis_name="c", subcore_axis_name="s")

    @functools.partial(
        pl.kernel,
        mesh=mesh,
        out_type=jax.ShapeDtypeStruct((G, HEADS, 2, N, _HC), jnp.float32),
        scratch_types=[
            pltpu.VMEM((N,), jnp.int32),
            pltpu.VMEM((N,), jnp.float32),
            pltpu.VMEM((N,), jnp.float32),
            pltpu.VMEM((_K,), jnp.int32),
            pltpu.VMEM((_K,), jnp.int32),
            pltpu.VMEM((_K,), jnp.int32),
            pltpu.VMEM((_K,), jnp.float32),
            pltpu.VMEM((_K, _HC), jnp.float32),
            pltpu.VMEM((_K, 16), jnp.float32),
            pltpu.VMEM_SHARED((N, _HC), jnp.float32),
            pltpu.VMEM_SHARED((N, 16), jnp.float32),
            pltpu.SemaphoreType.DMA,
        ],
        compiler_params=pltpu.CompilerParams(needs_layout_passes=False,
                                             use_tc_tiling_on_sc=False),
    )
    def k(x_hbm, src_hbm, dst_hbm, t00_hbm, t01_hbm, t10_hbm, t11_hbm,
          as0_hbm, as1_hbm, ad0_hbm, ad1_hbm, out_hbm,
          xg, asb, adb, srcb, dstb, msrcb, wb, rows, wbuf, accum, denom, sem):
        g = lax.axis_index("c")
        tid = lax.axis_index("s")
        iota16 = lax.iota(jnp.int32, 16)
        tqs = [t00_hbm, t01_hbm, t10_hbm, t11_hbm]
        logits = [(as0_hbm, ad0_hbm), (as1_hbm, ad1_hbm)]

        pltpu.sync_copy(x_hbm.at[g], xg)

        for h in range(HEADS):
            ash_hbm, adh_hbm = logits[h]
            pltpu.sync_copy(ash_hbm, asb)
            pltpu.sync_copy(adh_hbm, adb)
            for half in range(2):
                th_hbm = tqs[2 * h + half]
                first = half == 0

                def _zrow(r, _):
                    for cg in range(_HC // 16):
                        rows[r, pl.ds(cg * 16, 16)] = jnp.zeros((16,), jnp.float32)
                    wbuf[r, :] = jnp.zeros((16,), jnp.float32)
                    return 0
                lax.fori_loop(0, _K, _zrow, 0)
                for i in range((_NSLICES + _NT - 1) // _NT):
                    s = tid + _NT * i

                    @pl.when(s < _NSLICES)
                    def _():
                        nb = s * _NSL
                        pltpu.sync_copy(rows.at[pl.ds(0, _NSL)],
                                        accum.at[pl.ds(nb, _NSL)])
                        if first:
                            pltpu.sync_copy(wbuf.at[pl.ds(0, _NSL)],
                                            denom.at[pl.ds(nb, _NSL)])
                plsc.subcore_barrier()

                def _chunk(j, _):
                    c = tid + _NT * j

                    @pl.when(c < _CHUNKS)
                    def _():
                        base = c * _K
                        pltpu.sync_copy(src_hbm.at[pl.ds(base, _K)], srcb)
                        pltpu.sync_copy(dst_hbm.at[pl.ds(base, _K)], dstb)

                        def _grp(gi, _):
                            o = gi * 16
                            sv = srcb[pl.ds(o, 16)]
                            dv = dstb[pl.ds(o, 16)]
                            ms = plsc.load_gather(xg, [sv])
                            md = plsc.load_gather(xg, [dv])
                            av = (plsc.load_gather(asb, [ms])
                                  + plsc.load_gather(adb, [md]))
                            w = jnp.exp(jnp.maximum(av, 0.2 * av))
                            msrcb[pl.ds(o, 16)] = ms
                            wb[pl.ds(o, 16)] = w
                            return 0
                        lax.fori_loop(0, _K // 16, _grp, 0)

                        pltpu.async_copy(th_hbm.at[msrcb], rows, sem).wait()

                        def _scale_grp(gi, _):
                            wv = wb[pl.ds(gi * 16, 16)]

                            def _lane(j2, _):
                                s = jnp.sum(jnp.where(iota16 == j2, wv, 0.0))
                                e = gi * 16 + j2
                                for cg in range(_HC // 16):
                                    sl = pl.ds(cg * 16, 16)
                                    rows[e, sl] = rows[e, sl] * s
                                wbuf[e, :] = jnp.broadcast_to(s, (16,))
                                return 0
                            lax.fori_loop(0, 16, _lane, 0)
                            return 0
                        lax.fori_loop(0, _K // 16, _scale_grp, 0)

                        pltpu.sync_copy(rows, accum.at[dstb], add=True)
                        if first:
                            pltpu.sync_copy(wbuf, denom.at[dstb], add=True)
                    return 0
                lax.fori_loop(0, (_CHUNKS + _NT - 1) // _NT, _chunk, 0)
                plsc.subcore_barrier()

                for i in range((_NSLICES + _NT - 1) // _NT):
                    s = tid + _NT * i

                    @pl.when(s < _NSLICES)
                    def _():
                        nb = s * _NSL
                        pltpu.sync_copy(accum.at[pl.ds(nb, _NSL)],
                                        rows.at[pl.ds(0, _NSL)])
                        pltpu.sync_copy(denom.at[pl.ds(nb, _NSL)],
                                        wbuf.at[pl.ds(0, _NSL)])

                        def _nrm(nrow, _):
                            dv = wbuf[nrow, :]
                            dsum = jnp.sum(jnp.where(iota16 == 0, dv, 0.0))
                            rv = 1.0 / (jnp.broadcast_to(dsum, (16,)) + 1e-16)
                            for cg in range(_HC // 16):
                                sl = pl.ds(cg * 16, 16)
                                rows[nrow, sl] = rows[nrow, sl] * rv
                            return 0
                        lax.fori_loop(0, _NSL, _nrm, 0)
                        pltpu.sync_copy(rows.at[pl.ds(0, _NSL)],
                                        out_hbm.at[g, h, half, pl.ds(nb, _NSL)])
                plsc.subcore_barrier()

    return k(x2, src, dst, *tq, as0, as1, ad0, ad1)


def kernel(x, adjacency, emb_table, W, att_src, att_dst, bias):
    b, d, n = x.shape
    T, A = _tc_project(emb_table, W, att_src, att_dst)
    tq = [T[:, i * _HC:(i + 1) * _HC] for i in range(4)]
    x2 = x.reshape(G, N).astype(jnp.int32)
    src = adjacency[0].astype(jnp.int32)
    dst = adjacency[1].astype(jnp.int32)
    out5 = _sc_gat(x2, src, dst, tq,
                   A[:, 0], A[:, 1], A[:, 2], A[:, 3])
    out = jnp.transpose(out5, (0, 3, 1, 2, 4)).reshape(G * N, HEADS * OUT_CH)
    out = out + bias[None, :]
    return out.reshape(b, d, -1)
